# baseline (speedup 1.0000x reference)
import functools

import jax
import jax.numpy as jnp
from jax import lax
from jax.experimental import pallas as pl
from jax.experimental.pallas import tpu as pltpu
from jax.experimental.pallas import tpu_sc as plsc

N = 10000
E = 320000
HE = 2000
D = 128
NC = 2
NS = 16
NW = NC * NS
B = 125
CE = E // NW
NBLK = CE // B
NP = 10240
HEP = 2048

_SQRT_HALF = 0.7071067811865476


def _gelu(v):
    return 0.5 * v * (1.0 + lax.erf(v * _SQRT_HALF))


def _msg_body(x_ref, a_ref, wt_ref, out_ref):
    x = x_ref[...]
    send = a_ref[:, 0:1] + a_ref[:, 2:3]
    out_ref[...] = _gelu(
        jnp.dot(x, wt_ref[...], preferred_element_type=jnp.float32)) * send


def _colsum(part):
    ones = jnp.ones((NW, 1), jnp.float32)
    return lax.dot_general(part, ones, (((0,), (0,)), ((), ())),
                           preferred_element_type=jnp.float32)


def _edge_body(p_ref, dege_ref, degv_ref, out_ref, degv_col_ref):
    s = p_ref[:HEP, :] + p_ref[HEP:, :]
    deg = jnp.maximum(_colsum(dege_ref[...]), 1.0)
    out_ref[...] = s / deg
    degv_col_ref[...] = _colsum(degv_ref[...])


def _upd_body(x_ref, a_ref, wt_ref, b_ref, m0_ref, m1_ref, degv_ref, out_ref):
    x = x_ref[...]
    recv = a_ref[:, 0:1] + a_ref[:, 1:2]
    deg = jnp.maximum(degv_ref[...], 1.0)
    mi = (m0_ref[0] + m1_ref[0]) * (recv / deg)
    u = jnp.dot(x, wt_ref[...], preferred_element_type=jnp.float32) + b_ref[...]
    out_ref[...] = _gelu(u + mi)


def _stream_loop(table, idxg_v, idxs_v, buf0, buf1, acc, sem0, sem1,
                 ssem0, ssem1, nblk, hist_fn=None):
    def gather(i, buf, sem):
        return pltpu.make_async_copy(table.at[idxg_v.at[i]], buf, sem)

    def scat(i, buf, sem):
        return pltpu.make_async_copy(buf, acc.at[idxs_v.at[i]], sem)

    def scat_start(i, buf, sem):
        pltpu.async_copy(buf, acc.at[idxs_v.at[i]], sem, add=True)

    gather(0, buf0, sem0).start()
    gather(1, buf1, sem1).start()

    def step(t, carry):
        i = 2 * t
        if hist_fn is not None:
            hist_fn(t)
        gather(i, buf0, sem0).wait()
        scat_start(i, buf0, ssem0)
        gather(i + 1, buf1, sem1).wait()
        scat_start(i + 1, buf1, ssem1)
        scat(i, buf0, ssem0).wait()
        gather(i + 2, buf0, sem0).start()
        scat(i + 1, buf1, ssem1).wait()
        gather(i + 3, buf1, sem1).start()
        return carry

    lax.fori_loop(0, (nblk - 2) // 2, step, 0)
    i = nblk - 2
    gather(i, buf0, sem0).wait()
    scat_start(i, buf0, ssem0)
    gather(i + 1, buf1, sem1).wait()
    scat_start(i + 1, buf1, ssem1)
    scat(i, buf0, ssem0).wait()
    scat(i + 1, buf1, ssem1).wait()


def _dump_acc(acc, out, acc_rows, core, sub):
    nz = acc_rows // NS
    plsc.subcore_barrier()
    pltpu.sync_copy(acc.at[pl.ds(sub * nz, nz)],
                    out.at[pl.ds(core * acc_rows + sub * nz, nz)])


def _hist_vreg(idx1, hist, o):
    v = idx1[pl.ds(o, 16)]
    cnt, last = plsc.scan_count(v)
    plsc.addupdate_scatter(hist, [v], cnt.astype(jnp.float32), mask=last)


def _mesh():
    return plsc.VectorSubcoreMesh(core_axis_name="c", subcore_axis_name="s",
                                  num_cores=NC, num_subcores=NS)


_SC_PARAMS = pltpu.CompilerParams(needs_layout_passes=False)


@functools.lru_cache(maxsize=None)
def _make_sc_v2e():
    def body(table, idx_g, idx_s, flat_g, flat_s, zrows, zflat,
             out, degv_out, dege_out,
             idxg_v, idxs_v, buf0, buf1, idxg1, idxs1, hist_v, hist_e,
             acc, sem0, sem1, ssem0, ssem1):
        c = lax.axis_index("c")
        s = lax.axis_index("s")
        wid = s * NC + c
        row0 = wid * NBLK
        nz = HEP // NS
        pltpu.sync_copy(idx_g.at[pl.ds(row0, NBLK)], idxg_v)
        pltpu.sync_copy(idx_s.at[pl.ds(row0, NBLK)], idxs_v)
        pltpu.sync_copy(flat_g.at[pl.ds(wid * CE, CE)], idxg1)
        pltpu.sync_copy(flat_s.at[pl.ds(wid * CE, CE)], idxs1)
        pltpu.sync_copy(zflat.at[pl.ds(0, N)], hist_v)
        pltpu.sync_copy(zflat.at[pl.ds(0, HEP)], hist_e)
        pltpu.sync_copy(zrows.at[pl.ds(0, nz)], acc.at[pl.ds(s * nz, nz)])
        plsc.subcore_barrier()

        def hist_fn(t):
            base = pl.multiple_of(t * 256, 16)
            for j in range(16):
                _hist_vreg(idxg1, hist_v, base + j * 16)
                _hist_vreg(idxs1, hist_e, base + j * 16)

        _stream_loop(table, idxg_v, idxs_v, buf0, buf1, acc, sem0, sem1,
                     ssem0, ssem1, NBLK, hist_fn)
        for r in range(((NBLK - 2) // 2) * 16, CE // 16):
            _hist_vreg(idxg1, hist_v, r * 16)
            _hist_vreg(idxs1, hist_e, r * 16)
        pltpu.sync_copy(hist_v, degv_out.at[wid])
        pltpu.sync_copy(hist_e, dege_out.at[wid])
        _dump_acc(acc, out, HEP, c, s)

    return pl.kernel(
        body,
        out_type=(
            jax.ShapeDtypeStruct((NC * HEP, D), jnp.float32),
            jax.ShapeDtypeStruct((NW, N), jnp.float32),
            jax.ShapeDtypeStruct((NW, HEP), jnp.float32),
        ),
        mesh=_mesh(),
        compiler_params=_SC_PARAMS,
        scratch_types=[
            pltpu.VMEM((NBLK, B), jnp.int32),
            pltpu.VMEM((NBLK, B), jnp.int32),
            pltpu.VMEM((B, D), jnp.float32),
            pltpu.VMEM((B, D), jnp.float32),
            pltpu.VMEM((CE,), jnp.int32),
            pltpu.VMEM((CE,), jnp.int32),
            pltpu.VMEM((N,), jnp.float32),
            pltpu.VMEM((HEP,), jnp.float32),
            pltpu.VMEM_SHARED((HEP, D), jnp.float32),
            pltpu.SemaphoreType.DMA,
            pltpu.SemaphoreType.DMA,
            pltpu.SemaphoreType.DMA,
            pltpu.SemaphoreType.DMA,
        ],
    )


_HBLK = NBLK // 2


@functools.lru_cache(maxsize=None)
def _make_sc_e2v():
    def body(table, idx_g, idx_s, zrows,
             out,
             idxg_v, idxs_v, buf0, buf1, acc, sem0, sem1, ssem0, ssem1):
        c = lax.axis_index("c")
        s = lax.axis_index("s")
        wid = s * NC + c
        row0 = wid * NBLK
        nz = NP // NS
        pltpu.sync_copy(zrows, acc.at[pl.ds(s * nz, nz)])
        plsc.subcore_barrier()
        for h in range(2):
            pltpu.sync_copy(idx_g.at[pl.ds(row0 + h * _HBLK, _HBLK)], idxg_v)
            pltpu.sync_copy(idx_s.at[pl.ds(row0 + h * _HBLK, _HBLK)], idxs_v)
            _stream_loop(table, idxg_v, idxs_v, buf0, buf1, acc,
                         sem0, sem1, ssem0, ssem1, _HBLK)
        plsc.subcore_barrier()
        pltpu.sync_copy(acc.at[pl.ds(s * nz, nz)],
                        out.at[c, pl.ds(s * nz, nz)])

    return pl.kernel(
        body,
        out_type=jax.ShapeDtypeStruct((NC, NP, D), jnp.float32),
        mesh=_mesh(),
        compiler_params=_SC_PARAMS,
        scratch_types=[
            pltpu.VMEM((_HBLK, B), jnp.int32),
            pltpu.VMEM((_HBLK, B), jnp.int32),
            pltpu.VMEM((B, D), jnp.float32),
            pltpu.VMEM((B, D), jnp.float32),
            pltpu.VMEM_SHARED((NP, D), jnp.float32),
            pltpu.SemaphoreType.DMA,
            pltpu.SemaphoreType.DMA,
            pltpu.SemaphoreType.DMA,
            pltpu.SemaphoreType.DMA,
        ],
    )


_BN = 1000


def kernel(x, action, pair_v, pair_e, W_msg, W_upd, b_upd):
    idx_v = pair_v.reshape(E // B, B)
    idx_e = pair_e.reshape(E // B, B)
    zrows = jnp.zeros((NP // NS, D), jnp.float32)
    zflat = jnp.zeros((N,), jnp.float32)
    grid = (N // _BN,)

    m = pl.pallas_call(
        _msg_body,
        grid=grid,
        in_specs=[
            pl.BlockSpec((_BN, D), lambda i: (i, 0)),
            pl.BlockSpec((_BN, 3), lambda i: (i, 0)),
            pl.BlockSpec((D, D), lambda i: (0, 0)),
        ],
        out_specs=pl.BlockSpec((_BN, D), lambda i: (i, 0)),
        out_shape=jax.ShapeDtypeStruct((N, D), jnp.float32),
    )(x, action, W_msg.T)

    e_part, degv_part, dege_part = _make_sc_v2e()(
        m, idx_v, idx_e, pair_v, pair_e, zrows, zflat)

    e_feat, degv_col = pl.pallas_call(
        _edge_body,
        out_shape=(jax.ShapeDtypeStruct((HEP, D), jnp.float32),
                   jax.ShapeDtypeStruct((N, 1), jnp.float32)),
    )(e_part, dege_part, degv_part)

    m_part = _make_sc_e2v()(e_feat, idx_e, idx_v, zrows)

    h = pl.pallas_call(
        _upd_body,
        grid=grid,
        in_specs=[
            pl.BlockSpec((_BN, D), lambda i: (i, 0)),
            pl.BlockSpec((_BN, 3), lambda i: (i, 0)),
            pl.BlockSpec((D, D), lambda i: (0, 0)),
            pl.BlockSpec((1, D), lambda i: (0, 0)),
            pl.BlockSpec((1, _BN, D), lambda i: (0, i, 0)),
            pl.BlockSpec((1, _BN, D), lambda i: (1, i, 0)),
            pl.BlockSpec((_BN, 1), lambda i: (i, 0)),
        ],
        out_specs=pl.BlockSpec((_BN, D), lambda i: (i, 0)),
        out_shape=jax.ShapeDtypeStruct((N, D), jnp.float32),
    )(x, action, W_upd.T, b_upd.reshape(1, D), m_part, m_part, degv_col)

    return h


# speedup vs baseline: 1.2332x; 1.2332x over previous
import functools

import jax
import jax.numpy as jnp
from jax import lax
from jax.experimental import pallas as pl
from jax.experimental.pallas import tpu as pltpu
from jax.experimental.pallas import tpu_sc as plsc

N = 10000
E = 320000
HE = 2000
D = 128
NC = 2
NS = 16
NW = NC * NS
B = 125
CE = E // NW
NBLK = CE // B
NP = 10240
HEP = 2048

_SQRT_HALF = 0.7071067811865476


def _gelu(v):
    return 0.5 * v * (1.0 + lax.erf(v * _SQRT_HALF))


def _msg_body(x_ref, a_ref, wt_ref, out_ref):
    x = x_ref[...]
    send = a_ref[:, 0:1] + a_ref[:, 2:3]
    out_ref[...] = _gelu(
        jnp.dot(x, wt_ref[...], preferred_element_type=jnp.float32)) * send


def _colsum(part):
    ones = jnp.ones((NW, 1), jnp.float32)
    return lax.dot_general(part, ones, (((0,), (0,)), ((), ())),
                           preferred_element_type=jnp.float32)


def _edge_body(p_ref, dege_ref, degv_ref, out_ref, degv_col_ref):
    s = p_ref[:HEP, :] + p_ref[HEP:, :]
    deg = jnp.maximum(_colsum(dege_ref[...]), 1.0)
    out_ref[...] = s / deg
    degv_col_ref[...] = _colsum(degv_ref[...])


def _upd_body(x_ref, a_ref, wt_ref, b_ref, m0_ref, m1_ref, degv_ref, out_ref):
    x = x_ref[...]
    recv = a_ref[:, 0:1] + a_ref[:, 1:2]
    deg = jnp.maximum(degv_ref[...], 1.0)
    mi = (m0_ref[0] + m1_ref[0]) * (recv / deg)
    u = jnp.dot(x, wt_ref[...], preferred_element_type=jnp.float32) + b_ref[...]
    out_ref[...] = _gelu(u + mi)


def _stream_loop(table, idxg_v, idxs_v, buf0, buf1, acc, sem0, sem1,
                 ssem0, ssem1, nblk, hist_fn=None):
    def gather(i, buf, sem):
        return pltpu.make_async_copy(table.at[idxg_v.at[i]], buf, sem)

    def scat(i, buf, sem):
        return pltpu.make_async_copy(buf, acc.at[idxs_v.at[i]], sem)

    def scat_start(i, buf, sem):
        pltpu.async_copy(buf, acc.at[idxs_v.at[i]], sem, add=True)

    gather(0, buf0, sem0).start()
    gather(1, buf1, sem1).start()

    def step(t, carry):
        i = 2 * t
        if hist_fn is not None:
            hist_fn(t)
        gather(i, buf0, sem0).wait()
        pltpu.sync_copy(buf0, acc.at[idxs_v.at[i]], add=True)
        gather(i + 2, buf0, sem0).start()
        gather(i + 1, buf1, sem1).wait()
        pltpu.sync_copy(buf1, acc.at[idxs_v.at[i + 1]], add=True)
        gather(i + 3, buf1, sem1).start()
        return carry

    lax.fori_loop(0, (nblk - 2) // 2, step, 0)
    i = nblk - 2
    gather(i, buf0, sem0).wait()
    pltpu.sync_copy(buf0, acc.at[idxs_v.at[i]], add=True)
    gather(i + 1, buf1, sem1).wait()
    pltpu.sync_copy(buf1, acc.at[idxs_v.at[i + 1]], add=True)


def _dump_acc(acc, out, acc_rows, core, sub):
    nz = acc_rows // NS
    plsc.subcore_barrier()
    pltpu.sync_copy(acc.at[pl.ds(sub * nz, nz)],
                    out.at[pl.ds(core * acc_rows + sub * nz, nz)])


def _hist_vreg(idx1, hist, o):
    v = idx1[pl.ds(o, 16)]
    cnt, last = plsc.scan_count(v)
    plsc.addupdate_scatter(hist, [v], cnt.astype(jnp.float32), mask=last)


def _mesh():
    return plsc.VectorSubcoreMesh(core_axis_name="c", subcore_axis_name="s",
                                  num_cores=NC, num_subcores=NS)


_SC_PARAMS = pltpu.CompilerParams(needs_layout_passes=False)


@functools.lru_cache(maxsize=None)
def _make_sc_v2e():
    def body(table, idx_g, idx_s, flat_g, flat_s, zrows, zflat,
             out, degv_out, dege_out,
             idxg_v, idxs_v, buf0, buf1, idxg1, idxs1, hist_v, hist_e,
             acc, sem0, sem1, ssem0, ssem1):
        c = lax.axis_index("c")
        s = lax.axis_index("s")
        wid = s * NC + c
        row0 = wid * NBLK
        nz = HEP // NS
        pltpu.sync_copy(idx_g.at[pl.ds(row0, NBLK)], idxg_v)
        pltpu.sync_copy(idx_s.at[pl.ds(row0, NBLK)], idxs_v)
        pltpu.sync_copy(flat_g.at[pl.ds(wid * CE, CE)], idxg1)
        pltpu.sync_copy(flat_s.at[pl.ds(wid * CE, CE)], idxs1)
        pltpu.sync_copy(zflat.at[pl.ds(0, N)], hist_v)
        pltpu.sync_copy(zflat.at[pl.ds(0, HEP)], hist_e)
        pltpu.sync_copy(zrows.at[pl.ds(0, nz)], acc.at[pl.ds(s * nz, nz)])
        plsc.subcore_barrier()

        def hist_fn(t):
            base = pl.multiple_of(t * 256, 16)
            for j in range(16):
                _hist_vreg(idxg1, hist_v, base + j * 16)
                _hist_vreg(idxs1, hist_e, base + j * 16)

        _stream_loop(table, idxg_v, idxs_v, buf0, buf1, acc, sem0, sem1,
                     ssem0, ssem1, NBLK, hist_fn)
        for r in range(((NBLK - 2) // 2) * 16, CE // 16):
            _hist_vreg(idxg1, hist_v, r * 16)
            _hist_vreg(idxs1, hist_e, r * 16)
        pltpu.sync_copy(hist_v, degv_out.at[wid])
        pltpu.sync_copy(hist_e, dege_out.at[wid])
        _dump_acc(acc, out, HEP, c, s)

    return pl.kernel(
        body,
        out_type=(
            jax.ShapeDtypeStruct((NC * HEP, D), jnp.float32),
            jax.ShapeDtypeStruct((NW, N), jnp.float32),
            jax.ShapeDtypeStruct((NW, HEP), jnp.float32),
        ),
        mesh=_mesh(),
        compiler_params=_SC_PARAMS,
        scratch_types=[
            pltpu.VMEM((NBLK, B), jnp.int32),
            pltpu.VMEM((NBLK, B), jnp.int32),
            pltpu.VMEM((B, D), jnp.float32),
            pltpu.VMEM((B, D), jnp.float32),
            pltpu.VMEM((CE,), jnp.int32),
            pltpu.VMEM((CE,), jnp.int32),
            pltpu.VMEM((N,), jnp.float32),
            pltpu.VMEM((HEP,), jnp.float32),
            pltpu.VMEM_SHARED((HEP, D), jnp.float32),
            pltpu.SemaphoreType.DMA,
            pltpu.SemaphoreType.DMA,
            pltpu.SemaphoreType.DMA,
            pltpu.SemaphoreType.DMA,
        ],
    )


_HBLK = NBLK // 2


@functools.lru_cache(maxsize=None)
def _make_sc_e2v():
    def body(table, idx_g, idx_s, zrows,
             out,
             idxg_v, idxs_v, buf0, buf1, acc, sem0, sem1, ssem0, ssem1):
        c = lax.axis_index("c")
        s = lax.axis_index("s")
        wid = s * NC + c
        row0 = wid * NBLK
        nz = NP // NS
        pltpu.sync_copy(zrows, acc.at[pl.ds(s * nz, nz)])
        plsc.subcore_barrier()
        for h in range(2):
            pltpu.sync_copy(idx_g.at[pl.ds(row0 + h * _HBLK, _HBLK)], idxg_v)
            pltpu.sync_copy(idx_s.at[pl.ds(row0 + h * _HBLK, _HBLK)], idxs_v)
            _stream_loop(table, idxg_v, idxs_v, buf0, buf1, acc,
                         sem0, sem1, ssem0, ssem1, _HBLK)
        plsc.subcore_barrier()
        pltpu.sync_copy(acc.at[pl.ds(s * nz, nz)],
                        out.at[c, pl.ds(s * nz, nz)])

    return pl.kernel(
        body,
        out_type=jax.ShapeDtypeStruct((NC, NP, D), jnp.float32),
        mesh=_mesh(),
        compiler_params=_SC_PARAMS,
        scratch_types=[
            pltpu.VMEM((_HBLK, B), jnp.int32),
            pltpu.VMEM((_HBLK, B), jnp.int32),
            pltpu.VMEM((B, D), jnp.float32),
            pltpu.VMEM((B, D), jnp.float32),
            pltpu.VMEM_SHARED((NP, D), jnp.float32),
            pltpu.SemaphoreType.DMA,
            pltpu.SemaphoreType.DMA,
            pltpu.SemaphoreType.DMA,
            pltpu.SemaphoreType.DMA,
        ],
    )


_BN = 1000


def kernel(x, action, pair_v, pair_e, W_msg, W_upd, b_upd):
    idx_v = pair_v.reshape(E // B, B)
    idx_e = pair_e.reshape(E // B, B)
    zrows = jnp.zeros((NP // NS, D), jnp.float32)
    zflat = jnp.zeros((N,), jnp.float32)
    grid = (N // _BN,)

    m = pl.pallas_call(
        _msg_body,
        grid=grid,
        in_specs=[
            pl.BlockSpec((_BN, D), lambda i: (i, 0)),
            pl.BlockSpec((_BN, 3), lambda i: (i, 0)),
            pl.BlockSpec((D, D), lambda i: (0, 0)),
        ],
        out_specs=pl.BlockSpec((_BN, D), lambda i: (i, 0)),
        out_shape=jax.ShapeDtypeStruct((N, D), jnp.float32),
    )(x, action, W_msg.T)

    e_part, degv_part, dege_part = _make_sc_v2e()(
        m, idx_v, idx_e, pair_v, pair_e, zrows, zflat)

    e_feat, degv_col = pl.pallas_call(
        _edge_body,
        out_shape=(jax.ShapeDtypeStruct((HEP, D), jnp.float32),
                   jax.ShapeDtypeStruct((N, 1), jnp.float32)),
    )(e_part, dege_part, degv_part)

    m_part = _make_sc_e2v()(e_feat, idx_e, idx_v, zrows)

    h = pl.pallas_call(
        _upd_body,
        grid=grid,
        in_specs=[
            pl.BlockSpec((_BN, D), lambda i: (i, 0)),
            pl.BlockSpec((_BN, 3), lambda i: (i, 0)),
            pl.BlockSpec((D, D), lambda i: (0, 0)),
            pl.BlockSpec((1, D), lambda i: (0, 0)),
            pl.BlockSpec((1, _BN, D), lambda i: (0, i, 0)),
            pl.BlockSpec((1, _BN, D), lambda i: (1, i, 0)),
            pl.BlockSpec((_BN, 1), lambda i: (i, 0)),
        ],
        out_specs=pl.BlockSpec((_BN, D), lambda i: (i, 0)),
        out_shape=jax.ShapeDtypeStruct((N, D), jnp.float32),
    )(x, action, W_upd.T, b_upd.reshape(1, D), m_part, m_part, degv_col)

    return h


# speedup vs baseline: 1.2390x; 1.0047x over previous
import functools

import jax
import jax.numpy as jnp
from jax import lax
from jax.experimental import pallas as pl
from jax.experimental.pallas import tpu as pltpu
from jax.experimental.pallas import tpu_sc as plsc

N = 10000
E = 320000
HE = 2000
D = 128
NC = 2
NS = 16
NW = NC * NS
B = 125
CE = E // NW
NBLK = CE // B
NP = 10240
HEP = 2048

_SQRT_HALF = 0.7071067811865476


def _gelu(v):
    return 0.5 * v * (1.0 + lax.erf(v * _SQRT_HALF))


def _matmul_t(x, w):
    return lax.dot_general(x, w, (((1,), (1,)), ((), ())),
                           preferred_element_type=jnp.float32)


def _msg_body(x_ref, a_ref, w_ref, out_ref):
    x = x_ref[...]
    send = a_ref[:, 0:1] + a_ref[:, 2:3]
    out_ref[...] = _gelu(_matmul_t(x, w_ref[...])) * send


def _colsum(part):
    ones = jnp.ones((NW, 1), jnp.float32)
    return lax.dot_general(part, ones, (((0,), (0,)), ((), ())),
                           preferred_element_type=jnp.float32)


def _edge_body(p_ref, dege_ref, degv_ref, out_ref, degv_col_ref):
    s = p_ref[:HEP, :] + p_ref[HEP:, :]
    deg = jnp.maximum(_colsum(dege_ref[...]), 1.0)
    out_ref[...] = s / deg
    degv_col_ref[...] = _colsum(degv_ref[...])


def _upd_body(x_ref, a_ref, w_ref, b_ref, m0_ref, m1_ref, degv_ref, out_ref):
    x = x_ref[...]
    recv = a_ref[:, 0:1] + a_ref[:, 1:2]
    deg = jnp.maximum(degv_ref[...], 1.0)
    mi = (m0_ref[0] + m1_ref[0]) * (recv / deg)
    u = _matmul_t(x, w_ref[...]) + b_ref[...]
    out_ref[...] = _gelu(u + mi)


def _stream_loop(table, idxg_v, idxs_v, buf0, buf1, acc, sem0, sem1,
                 ssem0, ssem1, nblk, hist_fn=None):
    def gather(i, buf, sem):
        return pltpu.make_async_copy(table.at[idxg_v.at[i]], buf, sem)

    def scat(i, buf, sem):
        return pltpu.make_async_copy(buf, acc.at[idxs_v.at[i]], sem)

    def scat_start(i, buf, sem):
        pltpu.async_copy(buf, acc.at[idxs_v.at[i]], sem, add=True)

    gather(0, buf0, sem0).start()
    gather(1, buf1, sem1).start()

    def step(t, carry):
        i = 2 * t
        if hist_fn is not None:
            hist_fn(t)
        gather(i, buf0, sem0).wait()
        pltpu.sync_copy(buf0, acc.at[idxs_v.at[i]], add=True)
        gather(i + 2, buf0, sem0).start()
        gather(i + 1, buf1, sem1).wait()
        pltpu.sync_copy(buf1, acc.at[idxs_v.at[i + 1]], add=True)
        gather(i + 3, buf1, sem1).start()
        return carry

    lax.fori_loop(0, (nblk - 2) // 2, step, 0)
    i = nblk - 2
    gather(i, buf0, sem0).wait()
    pltpu.sync_copy(buf0, acc.at[idxs_v.at[i]], add=True)
    gather(i + 1, buf1, sem1).wait()
    pltpu.sync_copy(buf1, acc.at[idxs_v.at[i + 1]], add=True)


def _dump_acc(acc, out, acc_rows, core, sub):
    nz = acc_rows // NS
    plsc.subcore_barrier()
    pltpu.sync_copy(acc.at[pl.ds(sub * nz, nz)],
                    out.at[pl.ds(core * acc_rows + sub * nz, nz)])


def _hist_vreg(idx1, hist, o):
    v = idx1[pl.ds(o, 16)]
    cnt, last = plsc.scan_count(v)
    plsc.addupdate_scatter(hist, [v], cnt.astype(jnp.float32), mask=last)


def _mesh():
    return plsc.VectorSubcoreMesh(core_axis_name="c", subcore_axis_name="s",
                                  num_cores=NC, num_subcores=NS)


_SC_PARAMS = pltpu.CompilerParams(needs_layout_passes=False,
                                  disable_bounds_checks=True,
                                  disable_semaphore_checks=True)


@functools.lru_cache(maxsize=None)
def _make_sc_v2e():
    def body(table, idx_g, idx_s, flat_g, flat_s, zrows, zflat,
             out, degv_out, dege_out,
             idxg_v, idxs_v, buf0, buf1, idxg1, idxs1, hist_v, hist_e,
             acc, sem0, sem1, ssem0, ssem1):
        c = lax.axis_index("c")
        s = lax.axis_index("s")
        wid = s * NC + c
        row0 = wid * NBLK
        nz = HEP // NS
        pltpu.sync_copy(idx_g.at[pl.ds(row0, NBLK)], idxg_v)
        pltpu.sync_copy(idx_s.at[pl.ds(row0, NBLK)], idxs_v)
        pltpu.sync_copy(flat_g.at[pl.ds(wid * CE, CE)], idxg1)
        pltpu.sync_copy(flat_s.at[pl.ds(wid * CE, CE)], idxs1)
        pltpu.sync_copy(zflat.at[pl.ds(0, N)], hist_v)
        pltpu.sync_copy(zflat.at[pl.ds(0, HEP)], hist_e)
        pltpu.sync_copy(zrows.at[pl.ds(0, nz)], acc.at[pl.ds(s * nz, nz)])
        plsc.subcore_barrier()

        def hist_fn(t):
            base = pl.multiple_of(t * 256, 16)
            for j in range(16):
                _hist_vreg(idxg1, hist_v, base + j * 16)
                _hist_vreg(idxs1, hist_e, base + j * 16)

        _stream_loop(table, idxg_v, idxs_v, buf0, buf1, acc, sem0, sem1,
                     ssem0, ssem1, NBLK, hist_fn)
        for r in range(((NBLK - 2) // 2) * 16, CE // 16):
            _hist_vreg(idxg1, hist_v, r * 16)
            _hist_vreg(idxs1, hist_e, r * 16)
        pltpu.sync_copy(hist_v, degv_out.at[wid])
        pltpu.sync_copy(hist_e, dege_out.at[wid])
        _dump_acc(acc, out, HEP, c, s)

    return pl.kernel(
        body,
        out_type=(
            jax.ShapeDtypeStruct((NC * HEP, D), jnp.float32),
            jax.ShapeDtypeStruct((NW, N), jnp.float32),
            jax.ShapeDtypeStruct((NW, HEP), jnp.float32),
        ),
        mesh=_mesh(),
        compiler_params=_SC_PARAMS,
        scratch_types=[
            pltpu.VMEM((NBLK, B), jnp.int32),
            pltpu.VMEM((NBLK, B), jnp.int32),
            pltpu.VMEM((B, D), jnp.float32),
            pltpu.VMEM((B, D), jnp.float32),
            pltpu.VMEM((CE,), jnp.int32),
            pltpu.VMEM((CE,), jnp.int32),
            pltpu.VMEM((N,), jnp.float32),
            pltpu.VMEM((HEP,), jnp.float32),
            pltpu.VMEM_SHARED((HEP, D), jnp.float32),
            pltpu.SemaphoreType.DMA,
            pltpu.SemaphoreType.DMA,
            pltpu.SemaphoreType.DMA,
            pltpu.SemaphoreType.DMA,
        ],
    )


_HBLK = NBLK // 2


@functools.lru_cache(maxsize=None)
def _make_sc_e2v():
    def body(table, idx_g, idx_s, zrows,
             out,
             idxg_v, idxs_v, buf0, buf1, acc, sem0, sem1, ssem0, ssem1):
        c = lax.axis_index("c")
        s = lax.axis_index("s")
        wid = s * NC + c
        row0 = wid * NBLK
        nz = NP // NS
        pltpu.sync_copy(zrows, acc.at[pl.ds(s * nz, nz)])
        plsc.subcore_barrier()
        for h in range(2):
            pltpu.sync_copy(idx_g.at[pl.ds(row0 + h * _HBLK, _HBLK)], idxg_v)
            pltpu.sync_copy(idx_s.at[pl.ds(row0 + h * _HBLK, _HBLK)], idxs_v)
            _stream_loop(table, idxg_v, idxs_v, buf0, buf1, acc,
                         sem0, sem1, ssem0, ssem1, _HBLK)
        plsc.subcore_barrier()
        pltpu.sync_copy(acc.at[pl.ds(s * nz, nz)],
                        out.at[c, pl.ds(s * nz, nz)])

    return pl.kernel(
        body,
        out_type=jax.ShapeDtypeStruct((NC, NP, D), jnp.float32),
        mesh=_mesh(),
        compiler_params=_SC_PARAMS,
        scratch_types=[
            pltpu.VMEM((_HBLK, B), jnp.int32),
            pltpu.VMEM((_HBLK, B), jnp.int32),
            pltpu.VMEM((B, D), jnp.float32),
            pltpu.VMEM((B, D), jnp.float32),
            pltpu.VMEM_SHARED((NP, D), jnp.float32),
            pltpu.SemaphoreType.DMA,
            pltpu.SemaphoreType.DMA,
            pltpu.SemaphoreType.DMA,
            pltpu.SemaphoreType.DMA,
        ],
    )


_BN = 1000


def kernel(x, action, pair_v, pair_e, W_msg, W_upd, b_upd):
    idx_v = pair_v.reshape(E // B, B)
    idx_e = pair_e.reshape(E // B, B)
    zrows = jnp.zeros((NP // NS, D), jnp.float32)
    zflat = jnp.zeros((N,), jnp.float32)
    grid = (N // _BN,)

    m = pl.pallas_call(
        _msg_body,
        grid=grid,
        in_specs=[
            pl.BlockSpec((_BN, D), lambda i: (i, 0)),
            pl.BlockSpec((_BN, 3), lambda i: (i, 0)),
            pl.BlockSpec((D, D), lambda i: (0, 0)),
        ],
        out_specs=pl.BlockSpec((_BN, D), lambda i: (i, 0)),
        out_shape=jax.ShapeDtypeStruct((N, D), jnp.float32),
    )(x, action, W_msg)

    e_part, degv_part, dege_part = _make_sc_v2e()(
        m, idx_v, idx_e, pair_v, pair_e, zrows, zflat)

    e_feat, degv_col = pl.pallas_call(
        _edge_body,
        out_shape=(jax.ShapeDtypeStruct((HEP, D), jnp.float32),
                   jax.ShapeDtypeStruct((N, 1), jnp.float32)),
    )(e_part, dege_part, degv_part)

    m_part = _make_sc_e2v()(e_feat, idx_e, idx_v, zrows)

    h = pl.pallas_call(
        _upd_body,
        grid=grid,
        in_specs=[
            pl.BlockSpec((_BN, D), lambda i: (i, 0)),
            pl.BlockSpec((_BN, 3), lambda i: (i, 0)),
            pl.BlockSpec((D, D), lambda i: (0, 0)),
            pl.BlockSpec((1, D), lambda i: (0, 0)),
            pl.BlockSpec((1, _BN, D), lambda i: (0, i, 0)),
            pl.BlockSpec((1, _BN, D), lambda i: (1, i, 0)),
            pl.BlockSpec((_BN, 1), lambda i: (i, 0)),
        ],
        out_specs=pl.BlockSpec((_BN, D), lambda i: (i, 0)),
        out_shape=jax.ShapeDtypeStruct((N, D), jnp.float32),
    )(x, action, W_upd, b_upd.reshape(1, D), m_part, m_part, degv_col)

    return h


# speedup vs baseline: 1.2399x; 1.0007x over previous
import functools

import jax
import jax.numpy as jnp
from jax import lax
from jax.experimental import pallas as pl
from jax.experimental.pallas import tpu as pltpu
from jax.experimental.pallas import tpu_sc as plsc

N = 10000
E = 320000
HE = 2000
D = 128
NC = 2
NS = 16
NW = NC * NS
B = 125
CE = E // NW
NBLK = CE // B
NP = 10240
HEP = 2048

_SQRT_HALF = 0.7071067811865476


def _gelu(v):
    return 0.5 * v * (1.0 + lax.erf(v * _SQRT_HALF))


def _matmul_t(x, w):
    return lax.dot_general(x, w, (((1,), (1,)), ((), ())),
                           preferred_element_type=jnp.float32)


def _msg_body(x_ref, a_ref, w_ref, out_ref):
    x = x_ref[...]
    send = a_ref[:, 0:1] + a_ref[:, 2:3]
    out_ref[...] = _gelu(_matmul_t(x, w_ref[...])) * send


def _colsum(part):
    ones = jnp.ones((NW, 1), jnp.float32)
    return lax.dot_general(part, ones, (((0,), (0,)), ((), ())),
                           preferred_element_type=jnp.float32)


def _edge_body(p_ref, dege_ref, degv_ref, out_ref, degv_col_ref):
    s = p_ref[:HEP, :] + p_ref[HEP:, :]
    deg = jnp.maximum(_colsum(dege_ref[...]), 1.0)
    out_ref[...] = s / deg
    degv_col_ref[...] = _colsum(degv_ref[...])


def _upd_body(x_ref, a_ref, w_ref, b_ref, m0_ref, m1_ref, degv_ref, out_ref):
    x = x_ref[...]
    recv = a_ref[:, 0:1] + a_ref[:, 1:2]
    deg = jnp.maximum(degv_ref[...], 1.0)
    mi = (m0_ref[0] + m1_ref[0]) * (recv / deg)
    u = _matmul_t(x, w_ref[...]) + b_ref[...]
    out_ref[...] = _gelu(u + mi)


def _stream_loop(table, idxg_v, idxs_v, buf0, buf1, acc, sem0, sem1,
                 ssem0, ssem1, nblk, hist_fn=None):
    def gather(i, buf, sem):
        return pltpu.make_async_copy(table.at[idxg_v.at[i]], buf, sem)

    def scat(i, buf, sem):
        return pltpu.make_async_copy(buf, acc.at[idxs_v.at[i]], sem)

    def scat_start(i, buf, sem):
        pltpu.async_copy(buf, acc.at[idxs_v.at[i]], sem, add=True)

    gather(0, buf0, sem0).start()
    gather(1, buf1, sem1).start()

    def step(t, carry):
        i = 2 * t
        if hist_fn is not None:
            hist_fn(t)
        gather(i, buf0, sem0).wait()
        pltpu.sync_copy(buf0, acc.at[idxs_v.at[i]], add=True)
        gather(i + 2, buf0, sem0).start()
        gather(i + 1, buf1, sem1).wait()
        pltpu.sync_copy(buf1, acc.at[idxs_v.at[i + 1]], add=True)
        gather(i + 3, buf1, sem1).start()
        return carry

    lax.fori_loop(0, (nblk - 2) // 2, step, 0)
    i = nblk - 2
    gather(i, buf0, sem0).wait()
    pltpu.sync_copy(buf0, acc.at[idxs_v.at[i]], add=True)
    gather(i + 1, buf1, sem1).wait()
    pltpu.sync_copy(buf1, acc.at[idxs_v.at[i + 1]], add=True)


def _dump_acc(acc, out, acc_rows, core, sub):
    nz = acc_rows // NS
    plsc.subcore_barrier()
    pltpu.sync_copy(acc.at[pl.ds(sub * nz, nz)],
                    out.at[pl.ds(core * acc_rows + sub * nz, nz)])


def _hist_vreg(idx1, hist, o):
    v = idx1[pl.ds(o, 16)]
    cnt, last = plsc.scan_count(v)
    plsc.addupdate_scatter(hist, [v], cnt.astype(jnp.float32), mask=last)


def _mesh():
    return plsc.VectorSubcoreMesh(core_axis_name="c", subcore_axis_name="s",
                                  num_cores=NC, num_subcores=NS)


_SC_PARAMS = pltpu.CompilerParams(needs_layout_passes=False,
                                  disable_bounds_checks=True,
                                  disable_semaphore_checks=True,
                                  skip_device_barrier=True)


@functools.lru_cache(maxsize=None)
def _make_sc_v2e():
    def body(table, idx_g, idx_s, flat_g, flat_s, zrows, zflat,
             out, degv_out, dege_out,
             idxg_v, idxs_v, buf0, buf1, idxg1, idxs1, hist_v, hist_e,
             acc, sem0, sem1, ssem0, ssem1):
        c = lax.axis_index("c")
        s = lax.axis_index("s")
        wid = s * NC + c
        row0 = wid * NBLK
        nz = HEP // NS
        pltpu.sync_copy(idx_g.at[pl.ds(row0, NBLK)], idxg_v)
        pltpu.sync_copy(idx_s.at[pl.ds(row0, NBLK)], idxs_v)
        pltpu.sync_copy(flat_g.at[pl.ds(wid * CE, CE)], idxg1)
        pltpu.sync_copy(flat_s.at[pl.ds(wid * CE, CE)], idxs1)
        pltpu.sync_copy(zflat.at[pl.ds(0, N)], hist_v)
        pltpu.sync_copy(zflat.at[pl.ds(0, HEP)], hist_e)
        pltpu.sync_copy(zrows.at[pl.ds(0, nz)], acc.at[pl.ds(s * nz, nz)])
        plsc.subcore_barrier()

        def hist_fn(t):
            base = pl.multiple_of(t * 256, 16)
            for j in range(16):
                _hist_vreg(idxg1, hist_v, base + j * 16)
                _hist_vreg(idxs1, hist_e, base + j * 16)

        _stream_loop(table, idxg_v, idxs_v, buf0, buf1, acc, sem0, sem1,
                     ssem0, ssem1, NBLK, hist_fn)
        for r in range(((NBLK - 2) // 2) * 16, CE // 16):
            _hist_vreg(idxg1, hist_v, r * 16)
            _hist_vreg(idxs1, hist_e, r * 16)
        pltpu.sync_copy(hist_v, degv_out.at[wid])
        pltpu.sync_copy(hist_e, dege_out.at[wid])
        _dump_acc(acc, out, HEP, c, s)

    return pl.kernel(
        body,
        out_type=(
            jax.ShapeDtypeStruct((NC * HEP, D), jnp.float32),
            jax.ShapeDtypeStruct((NW, N), jnp.float32),
            jax.ShapeDtypeStruct((NW, HEP), jnp.float32),
        ),
        mesh=_mesh(),
        compiler_params=_SC_PARAMS,
        scratch_types=[
            pltpu.VMEM((NBLK, B), jnp.int32),
            pltpu.VMEM((NBLK, B), jnp.int32),
            pltpu.VMEM((B, D), jnp.float32),
            pltpu.VMEM((B, D), jnp.float32),
            pltpu.VMEM((CE,), jnp.int32),
            pltpu.VMEM((CE,), jnp.int32),
            pltpu.VMEM((N,), jnp.float32),
            pltpu.VMEM((HEP,), jnp.float32),
            pltpu.VMEM_SHARED((HEP, D), jnp.float32),
            pltpu.SemaphoreType.DMA,
            pltpu.SemaphoreType.DMA,
            pltpu.SemaphoreType.DMA,
            pltpu.SemaphoreType.DMA,
        ],
    )


_HBLK = NBLK // 2


@functools.lru_cache(maxsize=None)
def _make_sc_e2v():
    def body(table, idx_g, idx_s, zrows,
             out,
             idxg_v, idxs_v, buf0, buf1, acc, sem0, sem1, ssem0, ssem1):
        c = lax.axis_index("c")
        s = lax.axis_index("s")
        wid = s * NC + c
        row0 = wid * NBLK
        nz = NP // NS
        pltpu.sync_copy(zrows, acc.at[pl.ds(s * nz, nz)])
        plsc.subcore_barrier()
        for h in range(2):
            pltpu.sync_copy(idx_g.at[pl.ds(row0 + h * _HBLK, _HBLK)], idxg_v)
            pltpu.sync_copy(idx_s.at[pl.ds(row0 + h * _HBLK, _HBLK)], idxs_v)
            _stream_loop(table, idxg_v, idxs_v, buf0, buf1, acc,
                         sem0, sem1, ssem0, ssem1, _HBLK)
        plsc.subcore_barrier()
        pltpu.sync_copy(acc.at[pl.ds(s * nz, nz)],
                        out.at[c, pl.ds(s * nz, nz)])

    return pl.kernel(
        body,
        out_type=jax.ShapeDtypeStruct((NC, NP, D), jnp.float32),
        mesh=_mesh(),
        compiler_params=_SC_PARAMS,
        scratch_types=[
            pltpu.VMEM((_HBLK, B), jnp.int32),
            pltpu.VMEM((_HBLK, B), jnp.int32),
            pltpu.VMEM((B, D), jnp.float32),
            pltpu.VMEM((B, D), jnp.float32),
            pltpu.VMEM_SHARED((NP, D), jnp.float32),
            pltpu.SemaphoreType.DMA,
            pltpu.SemaphoreType.DMA,
            pltpu.SemaphoreType.DMA,
            pltpu.SemaphoreType.DMA,
        ],
    )


_BN = 1000


def kernel(x, action, pair_v, pair_e, W_msg, W_upd, b_upd):
    idx_v = pair_v.reshape(E // B, B)
    idx_e = pair_e.reshape(E // B, B)
    zrows = jnp.zeros((NP // NS, D), jnp.float32)
    zflat = jnp.zeros((N,), jnp.float32)
    grid = (N // _BN,)

    m = pl.pallas_call(
        _msg_body,
        grid=grid,
        in_specs=[
            pl.BlockSpec((_BN, D), lambda i: (i, 0)),
            pl.BlockSpec((_BN, 3), lambda i: (i, 0)),
            pl.BlockSpec((D, D), lambda i: (0, 0)),
        ],
        out_specs=pl.BlockSpec((_BN, D), lambda i: (i, 0)),
        out_shape=jax.ShapeDtypeStruct((N, D), jnp.float32),
    )(x, action, W_msg)

    e_part, degv_part, dege_part = _make_sc_v2e()(
        m, idx_v, idx_e, pair_v, pair_e, zrows, zflat)

    e_feat, degv_col = pl.pallas_call(
        _edge_body,
        out_shape=(jax.ShapeDtypeStruct((HEP, D), jnp.float32),
                   jax.ShapeDtypeStruct((N, 1), jnp.float32)),
    )(e_part, dege_part, degv_part)

    m_part = _make_sc_e2v()(e_feat, idx_e, idx_v, zrows)

    h = pl.pallas_call(
        _upd_body,
        grid=grid,
        in_specs=[
            pl.BlockSpec((_BN, D), lambda i: (i, 0)),
            pl.BlockSpec((_BN, 3), lambda i: (i, 0)),
            pl.BlockSpec((D, D), lambda i: (0, 0)),
            pl.BlockSpec((1, D), lambda i: (0, 0)),
            pl.BlockSpec((1, _BN, D), lambda i: (0, i, 0)),
            pl.BlockSpec((1, _BN, D), lambda i: (1, i, 0)),
            pl.BlockSpec((_BN, 1), lambda i: (i, 0)),
        ],
        out_specs=pl.BlockSpec((_BN, D), lambda i: (i, 0)),
        out_shape=jax.ShapeDtypeStruct((N, D), jnp.float32),
    )(x, action, W_upd, b_upd.reshape(1, D), m_part, m_part, degv_col)

    return h
